# baseline (speedup 1.0000x reference)
import functools

import jax
import jax.numpy as jnp
from jax import lax
from jax.experimental import pallas as pl
from jax.experimental.pallas import tpu as pltpu
from jax.experimental.pallas import tpu_sc as plsc

_ROWS = 320000
_COLS = 128

_NC = 2
_NS = 16
_NW = _NC * _NS
_SC_ROWS = 81920
_CH = 256
_SC_BASE = _ROWS - _SC_ROWS

_TC_ROWS = _ROWS - _SC_ROWS
_TC_NBLK = 8
_TC_BLOCK = _TC_ROWS // _TC_NBLK


def _tc_sum_kernel(x_ref, o_ref):
    i = pl.program_id(0)

    @pl.when(i == 0)
    def _init():
        o_ref[...] = jnp.zeros_like(o_ref)

    o_ref[...] += jnp.sum(x_ref[...], axis=0, keepdims=True)


def _tc_partial(x):
    return pl.pallas_call(
        _tc_sum_kernel,
        grid=(_TC_NBLK,),
        in_specs=[pl.BlockSpec((_TC_BLOCK, _COLS), lambda i: (i, 0))],
        out_specs=pl.BlockSpec((1, _COLS), lambda i: (0, 0)),
        out_shape=jax.ShapeDtypeStruct((1, _COLS), jnp.float32),
    )(x)


_ROWS_PER_W = _SC_ROWS // _NW
_NCHUNK = _ROWS_PER_W // _CH


@functools.partial(
    pl.kernel,
    mesh=plsc.VectorSubcoreMesh(
        core_axis_name="c", subcore_axis_name="s", num_cores=_NC
    ),
    out_type=jax.ShapeDtypeStruct((_NW, _COLS), jnp.float32),
    scratch_types=[
        pltpu.VMEM((_CH, _COLS), jnp.float32),
        pltpu.VMEM((_CH, _COLS), jnp.float32),
        pltpu.VMEM((_COLS,), jnp.float32),
        pltpu.SemaphoreType.DMA,
        pltpu.SemaphoreType.DMA,
    ],
)
def _sc_partial_kernel(x_hbm, out_hbm, buf0, buf1, acc_v, sem0, sem1):
    wid = lax.axis_index("s") * _NC + lax.axis_index("c")
    base = _SC_BASE + wid * _ROWS_PER_W

    bufs = (buf0, buf1)
    sems = (sem0, sem1)

    copies = [None, None]
    copies[0] = pltpu.async_copy(x_hbm.at[pl.ds(base, _CH)], buf0, sem0)

    accs = tuple(jnp.zeros((16,), jnp.float32) for _ in range(8))
    for k in range(_NCHUNK):
        cur = k % 2
        nxt = (k + 1) % 2
        if k + 1 < _NCHUNK:
            copies[nxt] = pltpu.async_copy(
                x_hbm.at[pl.ds(base + (k + 1) * _CH, _CH)], bufs[nxt], sems[nxt]
            )
        copies[cur].wait()
        buf = bufs[cur]

        def row_body(q, a):
            r = q * 4
            out = []
            for j in range(8):
                s01 = buf[r, pl.ds(j * 16, 16)] + buf[r + 1, pl.ds(j * 16, 16)]
                s23 = buf[r + 2, pl.ds(j * 16, 16)] + buf[r + 3, pl.ds(j * 16, 16)]
                out.append(a[j] + (s01 + s23))
            return tuple(out)

        accs = lax.fori_loop(0, _CH // 4, row_body, accs)

    for j in range(8):
        acc_v[pl.ds(j * 16, 16)] = accs[j]
    pltpu.sync_copy(acc_v, out_hbm.at[wid])


def kernel(x):
    sc_part = _sc_partial_kernel(x)
    tc_part = _tc_partial(x[:_TC_ROWS])
    return tc_part + jnp.sum(sc_part, axis=0, keepdims=True)


# speedup vs baseline: 1.4853x; 1.4853x over previous
import functools

import jax
import jax.numpy as jnp
from jax import lax
from jax.experimental import pallas as pl
from jax.experimental.pallas import tpu as pltpu
from jax.experimental.pallas import tpu_sc as plsc

_ROWS = 320000
_COLS = 128

_NC = 2
_NS = 16
_NW = _NC * _NS
_SC_ROWS = 320000
_CH = 200
_SC_BASE = _ROWS - _SC_ROWS

_TC_ROWS = _ROWS - _SC_ROWS
_TC_NBLK = 8
_TC_BLOCK = _TC_ROWS // _TC_NBLK


def _tc_sum_kernel(x_ref, o_ref):
    i = pl.program_id(0)

    @pl.when(i == 0)
    def _init():
        o_ref[...] = jnp.zeros_like(o_ref)

    o_ref[...] += jnp.sum(x_ref[...], axis=0, keepdims=True)


def _tc_partial(x):
    return pl.pallas_call(
        _tc_sum_kernel,
        grid=(_TC_NBLK,),
        in_specs=[pl.BlockSpec((_TC_BLOCK, _COLS), lambda i: (i, 0))],
        out_specs=pl.BlockSpec((1, _COLS), lambda i: (0, 0)),
        out_shape=jax.ShapeDtypeStruct((1, _COLS), jnp.float32),
    )(x)


_ROWS_PER_W = _SC_ROWS // _NW
_NCHUNK = _ROWS_PER_W // _CH


@functools.partial(
    pl.kernel,
    mesh=plsc.VectorSubcoreMesh(
        core_axis_name="c", subcore_axis_name="s", num_cores=_NC
    ),
    out_type=jax.ShapeDtypeStruct((_NW, _COLS), jnp.float32),
    scratch_types=[
        pltpu.VMEM((_CH, _COLS), jnp.float32),
        pltpu.VMEM((_CH, _COLS), jnp.float32),
        pltpu.VMEM((_COLS,), jnp.float32),
        pltpu.SemaphoreType.DMA,
        pltpu.SemaphoreType.DMA,
    ],
)
def _sc_partial_kernel(x_hbm, out_hbm, buf0, buf1, acc_v, sem0, sem1):
    wid = lax.axis_index("s") * _NC + lax.axis_index("c")
    base = _SC_BASE + wid * _ROWS_PER_W

    bufs = (buf0, buf1)
    sems = (sem0, sem1)

    copies = [None, None]
    copies[0] = pltpu.async_copy(x_hbm.at[pl.ds(base, _CH)], buf0, sem0)

    accs = tuple(jnp.zeros((16,), jnp.float32) for _ in range(8))
    for k in range(_NCHUNK):
        cur = k % 2
        nxt = (k + 1) % 2
        if k + 1 < _NCHUNK:
            copies[nxt] = pltpu.async_copy(
                x_hbm.at[pl.ds(base + (k + 1) * _CH, _CH)], bufs[nxt], sems[nxt]
            )
        copies[cur].wait()
        buf = bufs[cur]

        def row_body(q, a):
            r = q * 4
            out = []
            for j in range(8):
                s01 = buf[r, pl.ds(j * 16, 16)] + buf[r + 1, pl.ds(j * 16, 16)]
                s23 = buf[r + 2, pl.ds(j * 16, 16)] + buf[r + 3, pl.ds(j * 16, 16)]
                out.append(a[j] + (s01 + s23))
            return tuple(out)

        accs = lax.fori_loop(0, _CH // 4, row_body, accs)

    for j in range(8):
        acc_v[pl.ds(j * 16, 16)] = accs[j]
    pltpu.sync_copy(acc_v, out_hbm.at[wid])


def kernel(x):
    sc_part = _sc_partial_kernel(x)
    if _TC_ROWS:
        tc_part = _tc_partial(x[:_TC_ROWS])
        return tc_part + jnp.sum(sc_part, axis=0, keepdims=True)
    return jnp.sum(sc_part, axis=0, keepdims=True)


# speedup vs baseline: 2.0649x; 1.3903x over previous
import functools

import jax
import jax.numpy as jnp
from jax import lax
from jax.experimental import pallas as pl
from jax.experimental.pallas import tpu as pltpu
from jax.experimental.pallas import tpu_sc as plsc

_ROWS = 320000
_COLS = 128

_NC = 2
_NS = 16
_NW = _NC * _NS
_SC_ROWS = 81920
_CH = 256
_SC_BASE = _ROWS - _SC_ROWS

_TC_ROWS = _ROWS - _SC_ROWS
_TC_NBLK = 8
_TC_BLOCK = _TC_ROWS // _TC_NBLK


def _tc_sum_kernel(x_ref, o_ref):
    i = pl.program_id(0)

    @pl.when(i == 0)
    def _init():
        o_ref[...] = jnp.zeros_like(o_ref)

    o_ref[...] += jnp.sum(x_ref[...], axis=0, keepdims=True)


def _tc_partial(x):
    return pl.pallas_call(
        _tc_sum_kernel,
        grid=(_TC_NBLK,),
        in_specs=[pl.BlockSpec((_TC_BLOCK, _COLS), lambda i: (i, 0))],
        out_specs=pl.BlockSpec((1, _COLS), lambda i: (0, 0)),
        out_shape=jax.ShapeDtypeStruct((1, _COLS), jnp.float32),
    )(x)


_ROWS_PER_W = _SC_ROWS // _NW
_NCHUNK = _ROWS_PER_W // _CH


@functools.partial(
    pl.kernel,
    mesh=plsc.VectorSubcoreMesh(
        core_axis_name="c", subcore_axis_name="s", num_cores=_NC
    ),
    out_type=jax.ShapeDtypeStruct((_NW, _COLS), jnp.float32),
    scratch_types=[
        pltpu.VMEM((_CH, _COLS), jnp.float32),
        pltpu.VMEM((_CH, _COLS), jnp.float32),
        pltpu.VMEM((_COLS,), jnp.float32),
        pltpu.SemaphoreType.DMA,
        pltpu.SemaphoreType.DMA,
    ],
)
def _sc_partial_kernel(x_hbm, out_hbm, buf0, buf1, acc_v, sem0, sem1):
    wid = lax.axis_index("s") * _NC + lax.axis_index("c")
    base = _SC_BASE + wid * _ROWS_PER_W

    bufs = (buf0, buf1)
    sems = (sem0, sem1)

    copies = [None, None]
    copies[0] = pltpu.async_copy(x_hbm.at[pl.ds(base, _CH)], buf0, sem0)

    accs = tuple(jnp.zeros((16,), jnp.float32) for _ in range(8))
    for k in range(_NCHUNK):
        cur = k % 2
        nxt = (k + 1) % 2
        if k + 1 < _NCHUNK:
            copies[nxt] = pltpu.async_copy(
                x_hbm.at[pl.ds(base + (k + 1) * _CH, _CH)], bufs[nxt], sems[nxt]
            )
        copies[cur].wait()
        buf = bufs[cur]

        def row_body(q, a):
            r = q * 4
            out = []
            for j in range(8):
                s01 = buf[r, pl.ds(j * 16, 16)] + buf[r + 1, pl.ds(j * 16, 16)]
                s23 = buf[r + 2, pl.ds(j * 16, 16)] + buf[r + 3, pl.ds(j * 16, 16)]
                out.append(a[j] + (s01 + s23))
            return tuple(out)

        accs = lax.fori_loop(0, _CH // 4, row_body, accs)

    for j in range(8):
        acc_v[pl.ds(j * 16, 16)] = accs[j]
    pltpu.sync_copy(acc_v, out_hbm.at[wid])


def kernel(x):
    sc_part = _sc_partial_kernel(x)
    if _TC_ROWS:
        tc_part = _tc_partial(x)
        return tc_part + jnp.sum(sc_part, axis=0, keepdims=True)
    return jnp.sum(sc_part, axis=0, keepdims=True)


# speedup vs baseline: 2.0683x; 1.0016x over previous
import functools

import jax
import jax.numpy as jnp
from jax import lax
from jax.experimental import pallas as pl
from jax.experimental.pallas import tpu as pltpu
from jax.experimental.pallas import tpu_sc as plsc

_ROWS = 320000
_COLS = 128

_NC = 2
_NS = 16
_NW = _NC * _NS
_SC_ROWS = 81920
_CH = 256
_SC_BASE = _ROWS - _SC_ROWS

_TC_ROWS = _ROWS - _SC_ROWS
_TC_NBLK = 8
_TC_BLOCK = _TC_ROWS // _TC_NBLK


def _tc_sum_kernel(x_ref, o_ref):
    i = pl.program_id(0)

    @pl.when(i == 0)
    def _init():
        o_ref[...] = jnp.zeros_like(o_ref)

    o_ref[...] += jnp.sum(x_ref[...], axis=0, keepdims=True)


def _tc_partial(x):
    return pl.pallas_call(
        _tc_sum_kernel,
        grid=(_TC_NBLK,),
        in_specs=[pl.BlockSpec((_TC_BLOCK, _COLS), lambda i: (i, 0))],
        out_specs=pl.BlockSpec((1, _COLS), lambda i: (0, 0)),
        out_shape=jax.ShapeDtypeStruct((1, _COLS), jnp.float32),
        compiler_params=pltpu.CompilerParams(skip_device_barrier=True),
    )(x)


_ROWS_PER_W = _SC_ROWS // _NW
_NCHUNK = _ROWS_PER_W // _CH


@functools.partial(
    pl.kernel,
    mesh=plsc.VectorSubcoreMesh(
        core_axis_name="c", subcore_axis_name="s", num_cores=_NC
    ),
    out_type=jax.ShapeDtypeStruct((_NW, _COLS), jnp.float32),
    scratch_types=[
        pltpu.VMEM((_CH, _COLS), jnp.float32),
        pltpu.VMEM((_CH, _COLS), jnp.float32),
        pltpu.VMEM((_COLS,), jnp.float32),
        pltpu.SemaphoreType.DMA,
        pltpu.SemaphoreType.DMA,
    ],
)
def _sc_partial_kernel(x_hbm, out_hbm, buf0, buf1, acc_v, sem0, sem1):
    wid = lax.axis_index("s") * _NC + lax.axis_index("c")
    base = _SC_BASE + wid * _ROWS_PER_W

    bufs = (buf0, buf1)
    sems = (sem0, sem1)

    copies = [None, None]
    copies[0] = pltpu.async_copy(x_hbm.at[pl.ds(base, _CH)], buf0, sem0)

    accs = tuple(jnp.zeros((16,), jnp.float32) for _ in range(8))
    for k in range(_NCHUNK):
        cur = k % 2
        nxt = (k + 1) % 2
        if k + 1 < _NCHUNK:
            copies[nxt] = pltpu.async_copy(
                x_hbm.at[pl.ds(base + (k + 1) * _CH, _CH)], bufs[nxt], sems[nxt]
            )
        copies[cur].wait()
        buf = bufs[cur]

        def row_body(q, a):
            r = q * 4
            out = []
            for j in range(8):
                s01 = buf[r, pl.ds(j * 16, 16)] + buf[r + 1, pl.ds(j * 16, 16)]
                s23 = buf[r + 2, pl.ds(j * 16, 16)] + buf[r + 3, pl.ds(j * 16, 16)]
                out.append(a[j] + (s01 + s23))
            return tuple(out)

        accs = lax.fori_loop(0, _CH // 4, row_body, accs)

    for j in range(8):
        acc_v[pl.ds(j * 16, 16)] = accs[j]
    pltpu.sync_copy(acc_v, out_hbm.at[wid])


def kernel(x):
    sc_part = _sc_partial_kernel(x)
    if _TC_ROWS:
        tc_part = _tc_partial(x)
        return tc_part + jnp.sum(sc_part, axis=0, keepdims=True)
    return jnp.sum(sc_part, axis=0, keepdims=True)
